# baseline (speedup 1.0000x reference)
import functools

import jax
import jax.numpy as jnp
from jax import lax
from jax.experimental import pallas as pl
from jax.experimental.pallas import tpu as pltpu
from jax.experimental.pallas import tpu_sc as plsc

NC = 2
NS = 16
NW = NC * NS
K = 16


def _gather_body(bpw, uids, iids, utab, itab, uout, iout,
                 uids_s, iids_s, usem, isem):
    wid = lax.axis_index("s") * NC + lax.axis_index("c")
    base = wid * bpw
    pltpu.sync_copy(uids.at[pl.ds(base, bpw)], uids_s)
    pltpu.sync_copy(iids.at[pl.ds(base, bpw)], iids_s)

    nch = bpw // K

    def fire(j):
        off = j * K
        uvec = uids_s[pl.ds(off, K)]
        ivec = iids_s[pl.ds(off, K)]
        for t in range(K):
            pltpu.async_copy(utab.at[uvec[t]], uout.at[base + off + t], usem)
            pltpu.async_copy(itab.at[ivec[t]], iout.at[base + off + t], isem)

    def drain(j):
        off = j * K
        for t in range(K):
            pltpu.make_async_copy(utab.at[0], uout.at[base + off + t], usem).wait()
            pltpu.make_async_copy(itab.at[0], iout.at[base + off + t], isem).wait()

    fire(0)

    @pl.loop(0, nch - 1)
    def _(j):
        fire(j + 1)
        drain(j)

    drain(nch - 1)


def _sc_gather(user_ids, item_ids, user_table, item_table):
    B = user_ids.shape[0]
    D = user_table.shape[1]
    bpw = B // NW
    body = functools.partial(_gather_body, bpw)
    out2 = jax.ShapeDtypeStruct((B, D), jnp.float32)
    mesh = plsc.VectorSubcoreMesh(
        core_axis_name="c", subcore_axis_name="s", num_cores=NC, num_subcores=NS
    )
    k = pl.kernel(
        body,
        out_type=(out2, out2),
        mesh=mesh,
        scratch_types=[
            pltpu.VMEM((bpw,), jnp.int32),
            pltpu.VMEM((bpw,), jnp.int32),
            pltpu.SemaphoreType.DMA,
            pltpu.SemaphoreType.DMA,
        ],
    )
    return k(user_ids.astype(jnp.int32), item_ids.astype(jnp.int32),
             user_table, item_table)


def _mlp_body(u_ref, i_ref, w1u_ref, w1i_ref, b1_ref, w2_ref, b2_ref,
              w3_ref, b3_ref, o_ref):
    h = (
        jnp.dot(u_ref[...], w1u_ref[...], preferred_element_type=jnp.float32)
        + jnp.dot(i_ref[...], w1i_ref[...], preferred_element_type=jnp.float32)
        + b1_ref[...]
    )
    h = jnp.maximum(h, 0.0)
    h = jnp.dot(h, w2_ref[...], preferred_element_type=jnp.float32) + b2_ref[...]
    h = jnp.maximum(h, 0.0)
    z = jnp.sum(h * w3_ref[...], axis=1) + b3_ref[...]
    o_ref[...] = 1.0 / (1.0 + jnp.exp(-z))


def _tc_mlp(u, i, W1, b1, W2, b2, W3, b3):
    B, D = u.shape
    BT = 2048
    w1u = W1[:, :D].T
    w1i = W1[:, D:].T
    w2 = W2.T
    w3 = W3[0]
    grid = (B // BT,)
    return pl.pallas_call(
        _mlp_body,
        grid=grid,
        in_specs=[
            pl.BlockSpec((BT, D), lambda g: (g, 0)),
            pl.BlockSpec((BT, D), lambda g: (g, 0)),
            pl.BlockSpec(w1u.shape, lambda g: (0, 0)),
            pl.BlockSpec(w1i.shape, lambda g: (0, 0)),
            pl.BlockSpec(b1.shape, lambda g: (0,)),
            pl.BlockSpec(w2.shape, lambda g: (0, 0)),
            pl.BlockSpec(b2.shape, lambda g: (0,)),
            pl.BlockSpec(w3.shape, lambda g: (0,)),
            pl.BlockSpec(b3.shape, lambda g: (0,)),
        ],
        out_specs=pl.BlockSpec((BT,), lambda g: (g,)),
        out_shape=jax.ShapeDtypeStruct((B,), jnp.float32),
    )(u, i, w1u, w1i, b1, w2, b2, w3, b3)


def kernel(user_ids, item_ids, user_table, item_table, W1, b1, W2, b2, W3, b3):
    u, i = _sc_gather(user_ids, item_ids, user_table, item_table)
    return _tc_mlp(u, i, W1, b1, W2, b2, W3, b3)


# speedup vs baseline: 1.5289x; 1.5289x over previous
import functools

import jax
import jax.numpy as jnp
from jax import lax
from jax.experimental import pallas as pl
from jax.experimental.pallas import tpu as pltpu
from jax.experimental.pallas import tpu_sc as plsc

NC = 2
NS = 16
NW = NC * NS
CHUNK = 128
L = 16


def _gather_body(bpw, uids, iids, utab4, itab4, uout, iout,
                 uidx, iidx, ubuf, ibuf, sem):
    wid = lax.axis_index("s") * NC + lax.axis_index("c")
    base = wid * bpw
    nch = bpw // CHUNK
    pltpu.sync_copy(uids.at[pl.ds(base, bpw)], uidx)
    pltpu.sync_copy(iids.at[pl.ds(base, bpw)], iidx)
    for t in range(bpw // L):
        uidx[pl.ds(t * L, L)] = uidx[pl.ds(t * L, L)] >> 2
        iidx[pl.ds(t * L, L)] = iidx[pl.ds(t * L, L)] >> 2

    def fire(j):
        s = j % 2
        return (
            pltpu.async_copy(utab4.at[uidx.at[pl.ds(j * CHUNK, CHUNK)]],
                             ubuf.at[s], sem),
            pltpu.async_copy(itab4.at[iidx.at[pl.ds(j * CHUNK, CHUNK)]],
                             ibuf.at[s], sem),
        )

    pend = fire(0)
    for j in range(nch):
        nxt = fire(j + 1) if j + 1 < nch else None
        pend[0].wait()
        pend[1].wait()
        s = j % 2
        pltpu.sync_copy(ubuf.at[s], uout.at[pl.ds(base + j * CHUNK, CHUNK)])
        pltpu.sync_copy(ibuf.at[s], iout.at[pl.ds(base + j * CHUNK, CHUNK)])
        pend = nxt


def _sc_gather(user_ids, item_ids, utab4, itab4):
    B = user_ids.shape[0]
    bpw = B // NW
    body = functools.partial(_gather_body, bpw)
    out2 = jax.ShapeDtypeStruct((B, 128), jnp.float32)
    mesh = plsc.VectorSubcoreMesh(
        core_axis_name="c", subcore_axis_name="s", num_cores=NC, num_subcores=NS
    )
    k = pl.kernel(
        body,
        out_type=(out2, out2),
        mesh=mesh,
        scratch_types=[
            pltpu.VMEM((bpw,), jnp.int32),
            pltpu.VMEM((bpw,), jnp.int32),
            pltpu.VMEM((2, CHUNK, 128), jnp.float32),
            pltpu.VMEM((2, CHUNK, 128), jnp.float32),
            pltpu.SemaphoreType.DMA,
        ],
    )
    return k(user_ids.astype(jnp.int32), item_ids.astype(jnp.int32),
             utab4, itab4)


def _mlp_body(uw_ref, iw_ref, uid_ref, iid_ref, w1u_ref, w1i_ref, b1_ref,
              w2_ref, b2_ref, w3_ref, b3_ref, o_ref):
    bt = uw_ref.shape[0]
    col = lax.broadcasted_iota(jnp.int32, (bt, 128), 1) >> 5
    xu = jnp.where(col == (uid_ref[...][:, None] & 3), uw_ref[...], 0.0)
    xi = jnp.where(col == (iid_ref[...][:, None] & 3), iw_ref[...], 0.0)
    h = (
        jnp.dot(xu, w1u_ref[...], preferred_element_type=jnp.float32)
        + jnp.dot(xi, w1i_ref[...], preferred_element_type=jnp.float32)
        + b1_ref[...]
    )
    h = jnp.maximum(h, 0.0)
    h = jnp.dot(h, w2_ref[...], preferred_element_type=jnp.float32) + b2_ref[...]
    h = jnp.maximum(h, 0.0)
    z = jnp.sum(h * w3_ref[...], axis=1) + b3_ref[...]
    o_ref[...] = 1.0 / (1.0 + jnp.exp(-z))


def _tc_mlp(uw, iw, uid, iid, W1, b1, W2, b2, W3, b3):
    B = uw.shape[0]
    D = 32
    BT = 2048
    w1u = jnp.concatenate([W1[:, :D].T] * 4, axis=0)
    w1i = jnp.concatenate([W1[:, D:].T] * 4, axis=0)
    w2 = W2.T
    w3 = W3[0]
    grid = (B // BT,)
    return pl.pallas_call(
        _mlp_body,
        grid=grid,
        in_specs=[
            pl.BlockSpec((BT, 128), lambda g: (g, 0)),
            pl.BlockSpec((BT, 128), lambda g: (g, 0)),
            pl.BlockSpec((BT,), lambda g: (g,)),
            pl.BlockSpec((BT,), lambda g: (g,)),
            pl.BlockSpec(w1u.shape, lambda g: (0, 0)),
            pl.BlockSpec(w1i.shape, lambda g: (0, 0)),
            pl.BlockSpec(b1.shape, lambda g: (0,)),
            pl.BlockSpec(w2.shape, lambda g: (0, 0)),
            pl.BlockSpec(b2.shape, lambda g: (0,)),
            pl.BlockSpec(w3.shape, lambda g: (0,)),
            pl.BlockSpec(b3.shape, lambda g: (0,)),
        ],
        out_specs=pl.BlockSpec((BT,), lambda g: (g,)),
        out_shape=jax.ShapeDtypeStruct((B,), jnp.float32),
    )(uw, iw, uid, iid, w1u, w1i, b1, w2, b2, w3, b3)


def kernel(user_ids, item_ids, user_table, item_table, W1, b1, W2, b2, W3, b3):
    utab4 = user_table.reshape(user_table.shape[0] // 4, 128)
    itab4 = item_table.reshape(item_table.shape[0] // 4, 128)
    uw, iw = _sc_gather(user_ids, item_ids, utab4, itab4)
    return _tc_mlp(uw, iw, user_ids.astype(jnp.int32),
                   item_ids.astype(jnp.int32), W1, b1, W2, b2, W3, b3)


# speedup vs baseline: 2.6511x; 1.7341x over previous
import functools

import jax
import jax.numpy as jnp
from jax import lax
from jax.experimental import pallas as pl
from jax.experimental.pallas import tpu as pltpu
from jax.experimental.pallas import tpu_sc as plsc

NC = 2
NS = 16
NW = NC * NS
L = 16
WIN = 1024
NF = 32


def _scan_table(tabT, sids, cbuf, wbuf, out, base, bpw):
    n_users = tabT.shape[1]
    lasta = ((n_users - WIN) // 128) * 128

    first = sids[pl.ds(0, L)][0]
    last = sids[pl.ds(bpw - L, L)][L - 1]
    wlo0 = jnp.minimum((first >> 7) << 7, lasta)
    nwin = ((((last >> 7) << 7) - wlo0) >> 10) + 1

    def extract(wlo, whi, buf):
        for g in range(bpw // L):
            gids = sids[pl.ds(g * L, L)]
            gmin = gids[0]
            gmax = gids[L - 1]

            @pl.when(jnp.logical_and(gmin < whi, gmax >= wlo))
            def _():
                rel = gids - wlo
                m = jnp.logical_and(gids >= wlo, gids < whi)
                rows = lax.iota(jnp.int32, L) + (g * L)
                for f in range(NF):
                    fvec = jnp.full((L,), f, jnp.int32)
                    vals = plsc.load_gather(buf, [fvec, rel], mask=m)
                    plsc.store_scatter(cbuf, [rows, fvec], vals, mask=m)

    @pl.loop(0, nwin)
    def _(k):
        wlo = jnp.minimum(wlo0 + k * WIN, lasta)
        wlo = pl.multiple_of(wlo, 128)
        pltpu.sync_copy(tabT.at[:, pl.ds(wlo, WIN)], wbuf)
        extract(wlo, wlo + WIN, wbuf)

    pltpu.sync_copy(cbuf, out.at[pl.ds(base, bpw)])


def _scan_body(bpw, su, si, utabT, itabT, uout, iout, sids, cbuf, wbuf):
    wid = lax.axis_index("s") * NC + lax.axis_index("c")
    base = wid * bpw
    pltpu.sync_copy(su.at[pl.ds(base, bpw)], sids)
    _scan_table(utabT, sids, cbuf, wbuf, uout, base, bpw)
    pltpu.sync_copy(si.at[pl.ds(base, bpw)], sids)
    _scan_table(itabT, sids, cbuf, wbuf, iout, base, bpw)


def _sc_scan(su, si, utabT, itabT):
    B = su.shape[0]
    bpw = B // NW
    body = functools.partial(_scan_body, bpw)
    out2 = jax.ShapeDtypeStruct((B, NF), jnp.float32)
    mesh = plsc.VectorSubcoreMesh(
        core_axis_name="c", subcore_axis_name="s", num_cores=NC, num_subcores=NS
    )
    k = pl.kernel(
        body,
        out_type=(out2, out2),
        mesh=mesh,
        compiler_params=pltpu.CompilerParams(needs_layout_passes=False),
        scratch_types=[
            pltpu.VMEM((bpw,), jnp.int32),
            pltpu.VMEM((bpw, NF), jnp.float32),
            pltpu.VMEM((NF, WIN), jnp.float32),
        ],
    )
    return k(su, si, utabT, itabT)


def _unperm_body(bpw, semu, semi, invu, invi, uout, iout,
                 uidx, iidx, urows, irows, sem):
    wid = lax.axis_index("s") * NC + lax.axis_index("c")
    nchunk = bpw // 128
    pltpu.sync_copy(invu.at[pl.ds(wid * nchunk, nchunk)], uidx)
    pltpu.sync_copy(invi.at[pl.ds(wid * nchunk, nchunk)], iidx)
    copies = []
    for j in range(nchunk):
        copies.append(pltpu.async_copy(semu.at[uidx.at[j]], urows.at[j], sem))
        copies.append(pltpu.async_copy(semi.at[iidx.at[j]], irows.at[j], sem))
    for c in copies:
        c.wait()
    pltpu.sync_copy(urows, uout.at[pl.ds(wid * nchunk, nchunk)])
    pltpu.sync_copy(irows, iout.at[pl.ds(wid * nchunk, nchunk)])


def _sc_unpermute(semb_u, semb_i, inv_pu, inv_pi):
    B = semb_u.shape[0]
    D = semb_u.shape[1]
    bpw = B // NW
    nchunk = bpw // 128
    invu2 = inv_pu.reshape(B // 128, 128)
    invi2 = inv_pi.reshape(B // 128, 128)
    body = functools.partial(_unperm_body, bpw)
    out3 = jax.ShapeDtypeStruct((B // 128, 128, D), jnp.float32)
    mesh = plsc.VectorSubcoreMesh(
        core_axis_name="c", subcore_axis_name="s", num_cores=NC, num_subcores=NS
    )
    k = pl.kernel(
        body,
        out_type=(out3, out3),
        mesh=mesh,
        compiler_params=pltpu.CompilerParams(use_tc_tiling_on_sc=False),
        scratch_types=[
            pltpu.VMEM((nchunk, 128), jnp.int32),
            pltpu.VMEM((nchunk, 128), jnp.int32),
            pltpu.VMEM((nchunk, 128, D), jnp.float32),
            pltpu.VMEM((nchunk, 128, D), jnp.float32),
            pltpu.SemaphoreType.DMA,
        ],
    )
    u3, i3 = k(semb_u, semb_i, invu2, invi2)
    return u3.reshape(B, D), i3.reshape(B, D)


def _mlp_body(utailo, itailo, u_ref, i_ref, uid_ref, iid_ref, tu_ref, ti_ref,
              w1u_ref, w1i_ref, b1_ref, w2_ref, b2_ref,
              w3_ref, b3_ref, o_ref):
    bt = u_ref.shape[0]

    def fix(x, ids2, tail_ref, tailo):
        n = tail_ref.shape[0]
        idb = lax.broadcast_in_dim(ids2, (bt, n), (0, 1))
        rel = jnp.clip(idb - tailo, 0, n - 1)
        oh = (rel == lax.broadcasted_iota(jnp.int32, (bt, n), 1))
        tv = jnp.dot(oh.astype(jnp.float32), tail_ref[...],
                     preferred_element_type=jnp.float32)
        keep = lax.broadcast_in_dim(ids2 < tailo, (bt, x.shape[1]), (0, 1))
        return jnp.where(keep, x, tv)

    u = fix(u_ref[...], uid_ref[...], tu_ref, utailo)
    i = fix(i_ref[...], iid_ref[...], ti_ref, itailo)
    h = (
        jnp.dot(u, w1u_ref[...], preferred_element_type=jnp.float32)
        + jnp.dot(i, w1i_ref[...], preferred_element_type=jnp.float32)
        + b1_ref[...]
    )
    h = jnp.maximum(h, 0.0)
    h = jnp.dot(h, w2_ref[...], preferred_element_type=jnp.float32) + b2_ref[...]
    h = jnp.maximum(h, 0.0)
    z = jnp.sum(h * w3_ref[...], axis=1) + b3_ref[...]
    o_ref[...] = 1.0 / (1.0 + jnp.exp(-z))


def _tc_mlp(u, i, uid, iid, tail_u, tail_i, W1, b1, W2, b2, W3, b3):
    B, D = u.shape
    BT = 2048
    w1u = W1[:, :D].T
    w1i = W1[:, D:].T
    w2 = W2.T
    w3 = W3[0]
    grid = (B // BT,)
    body = functools.partial(_mlp_body, NUM_USERS_TAILO, NUM_ITEMS_TAILO)
    return pl.pallas_call(
        body,
        grid=grid,
        in_specs=[
            pl.BlockSpec((BT, D), lambda g: (g, 0)),
            pl.BlockSpec((BT, D), lambda g: (g, 0)),
            pl.BlockSpec((BT, 1), lambda g: (g, 0)),
            pl.BlockSpec((BT, 1), lambda g: (g, 0)),
            pl.BlockSpec(tail_u.shape, lambda g: (0, 0)),
            pl.BlockSpec(tail_i.shape, lambda g: (0, 0)),
            pl.BlockSpec(w1u.shape, lambda g: (0, 0)),
            pl.BlockSpec(w1i.shape, lambda g: (0, 0)),
            pl.BlockSpec(b1.shape, lambda g: (0,)),
            pl.BlockSpec(w2.shape, lambda g: (0, 0)),
            pl.BlockSpec(b2.shape, lambda g: (0,)),
            pl.BlockSpec(w3.shape, lambda g: (0,)),
            pl.BlockSpec(b3.shape, lambda g: (0,)),
        ],
        out_specs=pl.BlockSpec((BT,), lambda g: (g,)),
        out_shape=jax.ShapeDtypeStruct((B,), jnp.float32),
    )(u, i, uid.reshape(B, 1), iid.reshape(B, 1), tail_u, tail_i, w1u, w1i, b1, w2, b2, w3, b3)


NUM_USERS_TAILO = 999936
NUM_ITEMS_TAILO = 99968


def kernel(user_ids, item_ids, user_table, item_table, W1, b1, W2, b2, W3, b3):
    B = user_ids.shape[0]
    uid = user_ids.astype(jnp.int32)
    iid = item_ids.astype(jnp.int32)
    pos = lax.iota(jnp.int32, B)
    su, pu = lax.sort((uid, pos), num_keys=1)
    si, pi = lax.sort((iid, pos), num_keys=1)
    _, inv_pu = lax.sort((pu, pos), num_keys=1)
    _, inv_pi = lax.sort((pi, pos), num_keys=1)
    semb_u, semb_i = _sc_scan(su, si, user_table.T, item_table.T)
    u, i = _sc_unpermute(semb_u, semb_i, inv_pu, inv_pi)
    tail_u = user_table[NUM_USERS_TAILO:, :]
    tail_i = item_table[NUM_ITEMS_TAILO:, :]
    return _tc_mlp(u, i, uid, iid, tail_u, tail_i, W1, b1, W2, b2, W3, b3)


# speedup vs baseline: 4.4520x; 1.6793x over previous
import functools

import jax
import jax.numpy as jnp
from jax import lax
from jax.experimental import pallas as pl
from jax.experimental.pallas import tpu as pltpu
from jax.experimental.pallas import tpu_sc as plsc

NC = 2
NS = 16
NW = NC * NS
L = 16
WIN = 512
NF = 32


def _scan_table(tabT, sids, gbmin, gbmax, cbuf, wbuf0, wbuf1,
                sem0, sem1, out, base, bpw):
    n_users = tabT.shape[1]
    lasta = ((n_users - WIN) // 128) * 128
    ngrp = bpw // L

    i16 = lax.iota(jnp.int32, L)
    for h in range(ngrp // L):
        gbmin[pl.ds(h * L, L)] = plsc.load_gather(
            sids, [i16 * L + (h * L * L)])
        gbmax[pl.ds(h * L, L)] = plsc.load_gather(
            sids, [i16 * L + (h * L * L + L - 1)])

    first = sids[pl.ds(0, L)][0]
    last = sids[pl.ds(bpw - L, L)][L - 1]
    wlo0 = jnp.minimum((first >> 7) << 7, lasta)
    nwin = (((last >> 7) << 7) - wlo0) // WIN + 1
    nwin2 = ((nwin + 1) // 2) * 2

    def wstart(k):
        w = jnp.minimum(wlo0 + k * WIN, lasta)
        return pl.multiple_of(w, 128)

    def fire(k, buf, sem):
        pltpu.async_copy(tabT.at[:, pl.ds(wstart(k), WIN)], buf, sem)

    def drain(buf, sem):
        pltpu.make_async_copy(tabT.at[:, pl.ds(0, WIN)], buf, sem).wait()

    def extract(k, buf):
        wlo = wstart(k)
        whi = wlo + WIN
        gl = jnp.int32(0)
        gh = jnp.int32(0)
        for h in range(ngrp // L):
            mx = gbmax[pl.ds(h * L, L)]
            mn = gbmin[pl.ds(h * L, L)]
            gl = gl + plsc.all_reduce_population_count(mx < wlo)[0]
            gh = gh + plsc.all_reduce_population_count(mn < whi)[0]

        @pl.loop(gl, gh)
        def _(g):
            gids = sids[pl.ds(g * L, L)]
            rel = gids - wlo
            m = jnp.logical_and(gids >= wlo, gids < whi)
            rows = i16 + g * L
            for f in range(NF):
                fvec = jnp.full((L,), f, jnp.int32)
                vals = plsc.load_gather(buf, [fvec, rel], mask=m)
                plsc.store_scatter(cbuf, [rows, fvec], vals, mask=m)

    fire(0, wbuf0, sem0)
    fire(1, wbuf1, sem1)

    @pl.loop(0, nwin2, step=2)
    def _(j):
        drain(wbuf0, sem0)
        extract(j, wbuf0)
        fire(j + 2, wbuf0, sem0)
        drain(wbuf1, sem1)
        extract(j + 1, wbuf1)
        fire(j + 3, wbuf1, sem1)

    drain(wbuf0, sem0)
    drain(wbuf1, sem1)

    pltpu.sync_copy(cbuf, out.at[pl.ds(base, bpw)])


def _scan_body(bpw, su, si, utabT, itabT, uout, iout,
               sids, gbmin, gbmax, cbuf, wbuf0, wbuf1, sem0, sem1):
    wid = lax.axis_index("s") * NC + lax.axis_index("c")
    base = wid * bpw
    pltpu.sync_copy(su.at[pl.ds(base, bpw)], sids)
    _scan_table(utabT, sids, gbmin, gbmax, cbuf, wbuf0, wbuf1,
                sem0, sem1, uout, base, bpw)
    pltpu.sync_copy(si.at[pl.ds(base, bpw)], sids)
    _scan_table(itabT, sids, gbmin, gbmax, cbuf, wbuf0, wbuf1,
                sem0, sem1, iout, base, bpw)


def _sc_scan(su, si, utabT, itabT):
    B = su.shape[0]
    bpw = B // NW
    body = functools.partial(_scan_body, bpw)
    out2 = jax.ShapeDtypeStruct((B, NF), jnp.float32)
    mesh = plsc.VectorSubcoreMesh(
        core_axis_name="c", subcore_axis_name="s", num_cores=NC, num_subcores=NS
    )
    k = pl.kernel(
        body,
        out_type=(out2, out2),
        mesh=mesh,
        compiler_params=pltpu.CompilerParams(needs_layout_passes=False),
        scratch_types=[
            pltpu.VMEM((bpw,), jnp.int32),
            pltpu.VMEM((bpw // L,), jnp.int32),
            pltpu.VMEM((bpw // L,), jnp.int32),
            pltpu.VMEM((bpw, NF), jnp.float32),
            pltpu.VMEM((NF, WIN), jnp.float32),
            pltpu.VMEM((NF, WIN), jnp.float32),
            pltpu.SemaphoreType.DMA,
            pltpu.SemaphoreType.DMA,
        ],
    )
    return k(su, si, utabT, itabT)


def _unperm_body(bpw, semu, semi, invu, invi, uout, iout,
                 uidx, iidx, urows, irows, sem):
    wid = lax.axis_index("s") * NC + lax.axis_index("c")
    nchunk = bpw // 128
    pltpu.sync_copy(invu.at[pl.ds(wid * nchunk, nchunk)], uidx)
    pltpu.sync_copy(invi.at[pl.ds(wid * nchunk, nchunk)], iidx)
    copies = []
    for j in range(nchunk):
        copies.append(pltpu.async_copy(semu.at[uidx.at[j]], urows.at[j], sem))
        copies.append(pltpu.async_copy(semi.at[iidx.at[j]], irows.at[j], sem))
    for c in copies:
        c.wait()
    pltpu.sync_copy(urows, uout.at[pl.ds(wid * nchunk, nchunk)])
    pltpu.sync_copy(irows, iout.at[pl.ds(wid * nchunk, nchunk)])


def _sc_unpermute(semb_u, semb_i, inv_pu, inv_pi):
    B = semb_u.shape[0]
    D = semb_u.shape[1]
    bpw = B // NW
    nchunk = bpw // 128
    invu2 = inv_pu.reshape(B // 128, 128)
    invi2 = inv_pi.reshape(B // 128, 128)
    body = functools.partial(_unperm_body, bpw)
    out3 = jax.ShapeDtypeStruct((B // 128, 128, D), jnp.float32)
    mesh = plsc.VectorSubcoreMesh(
        core_axis_name="c", subcore_axis_name="s", num_cores=NC, num_subcores=NS
    )
    k = pl.kernel(
        body,
        out_type=(out3, out3),
        mesh=mesh,
        compiler_params=pltpu.CompilerParams(use_tc_tiling_on_sc=False),
        scratch_types=[
            pltpu.VMEM((nchunk, 128), jnp.int32),
            pltpu.VMEM((nchunk, 128), jnp.int32),
            pltpu.VMEM((nchunk, 128, D), jnp.float32),
            pltpu.VMEM((nchunk, 128, D), jnp.float32),
            pltpu.SemaphoreType.DMA,
        ],
    )
    u3, i3 = k(semb_u, semb_i, invu2, invi2)
    return u3.reshape(B, D), i3.reshape(B, D)


def _mlp_body(utailo, itailo, u_ref, i_ref, uid_ref, iid_ref, tu_ref, ti_ref,
              w1u_ref, w1i_ref, b1_ref, w2_ref, b2_ref,
              w3_ref, b3_ref, o_ref):
    bt = u_ref.shape[0]

    def fix(x, ids2, tail_ref, tailo):
        n = tail_ref.shape[0]
        idb = lax.broadcast_in_dim(ids2, (bt, n), (0, 1))
        rel = jnp.clip(idb - tailo, 0, n - 1)
        oh = (rel == lax.broadcasted_iota(jnp.int32, (bt, n), 1))
        tv = jnp.dot(oh.astype(jnp.float32), tail_ref[...],
                     preferred_element_type=jnp.float32)
        keep = lax.broadcast_in_dim(ids2 < tailo, (bt, x.shape[1]), (0, 1))
        return jnp.where(keep, x, tv)

    u = fix(u_ref[...], uid_ref[...], tu_ref, utailo)
    i = fix(i_ref[...], iid_ref[...], ti_ref, itailo)
    h = (
        jnp.dot(u, w1u_ref[...], preferred_element_type=jnp.float32)
        + jnp.dot(i, w1i_ref[...], preferred_element_type=jnp.float32)
        + b1_ref[...]
    )
    h = jnp.maximum(h, 0.0)
    h = jnp.dot(h, w2_ref[...], preferred_element_type=jnp.float32) + b2_ref[...]
    h = jnp.maximum(h, 0.0)
    z = jnp.sum(h * w3_ref[...], axis=1) + b3_ref[...]
    o_ref[...] = 1.0 / (1.0 + jnp.exp(-z))


def _tc_mlp(u, i, uid, iid, tail_u, tail_i, W1, b1, W2, b2, W3, b3):
    B, D = u.shape
    BT = 2048
    w1u = W1[:, :D].T
    w1i = W1[:, D:].T
    w2 = W2.T
    w3 = W3[0]
    grid = (B // BT,)
    body = functools.partial(_mlp_body, NUM_USERS_TAILO, NUM_ITEMS_TAILO)
    return pl.pallas_call(
        body,
        grid=grid,
        in_specs=[
            pl.BlockSpec((BT, D), lambda g: (g, 0)),
            pl.BlockSpec((BT, D), lambda g: (g, 0)),
            pl.BlockSpec((BT, 1), lambda g: (g, 0)),
            pl.BlockSpec((BT, 1), lambda g: (g, 0)),
            pl.BlockSpec(tail_u.shape, lambda g: (0, 0)),
            pl.BlockSpec(tail_i.shape, lambda g: (0, 0)),
            pl.BlockSpec(w1u.shape, lambda g: (0, 0)),
            pl.BlockSpec(w1i.shape, lambda g: (0, 0)),
            pl.BlockSpec(b1.shape, lambda g: (0,)),
            pl.BlockSpec(w2.shape, lambda g: (0, 0)),
            pl.BlockSpec(b2.shape, lambda g: (0,)),
            pl.BlockSpec(w3.shape, lambda g: (0,)),
            pl.BlockSpec(b3.shape, lambda g: (0,)),
        ],
        out_specs=pl.BlockSpec((BT,), lambda g: (g,)),
        out_shape=jax.ShapeDtypeStruct((B,), jnp.float32),
    )(u, i, uid.reshape(B, 1), iid.reshape(B, 1), tail_u, tail_i, w1u, w1i, b1, w2, b2, w3, b3)


NUM_USERS_TAILO = 999936
NUM_ITEMS_TAILO = 99968


def kernel(user_ids, item_ids, user_table, item_table, W1, b1, W2, b2, W3, b3):
    B = user_ids.shape[0]
    uid = user_ids.astype(jnp.int32)
    iid = item_ids.astype(jnp.int32)
    pos = lax.iota(jnp.int32, B)
    su, pu = lax.sort((uid, pos), num_keys=1)
    si, pi = lax.sort((iid, pos), num_keys=1)
    _, inv_pu = lax.sort((pu, pos), num_keys=1)
    _, inv_pi = lax.sort((pi, pos), num_keys=1)
    semb_u, semb_i = _sc_scan(su, si, user_table.T, item_table.T)
    u, i = _sc_unpermute(semb_u, semb_i, inv_pu, inv_pi)
    tail_u = user_table[NUM_USERS_TAILO:, :]
    tail_i = item_table[NUM_ITEMS_TAILO:, :]
    return _tc_mlp(u, i, uid, iid, tail_u, tail_i, W1, b1, W2, b2, W3, b3)


# speedup vs baseline: 4.5195x; 1.0152x over previous
import functools

import jax
import jax.numpy as jnp
from jax import lax
from jax.experimental import pallas as pl
from jax.experimental.pallas import tpu as pltpu
from jax.experimental.pallas import tpu_sc as plsc

NC = 2
NS = 16
NW = NC * NS
L = 16
WIN = 768
NF = 32


def _scan_table(tabT, sids, gbmin, gbmax, cbuf, wbuf0, wbuf1,
                sem0, sem1, out, base, bpw):
    n_users = tabT.shape[1]
    lasta = ((n_users - WIN) // 128) * 128
    ngrp = bpw // L

    i16 = lax.iota(jnp.int32, L)
    for h in range(ngrp // L):
        gbmin[pl.ds(h * L, L)] = plsc.load_gather(
            sids, [i16 * L + (h * L * L)])
        gbmax[pl.ds(h * L, L)] = plsc.load_gather(
            sids, [i16 * L + (h * L * L + L - 1)])

    first = sids[pl.ds(0, L)][0]
    last = sids[pl.ds(bpw - L, L)][L - 1]
    wlo0 = jnp.minimum((first >> 7) << 7, lasta)
    nwin = (((last >> 7) << 7) - wlo0) // WIN + 1
    nwin2 = ((nwin + 1) // 2) * 2

    def wstart(k):
        w = jnp.minimum(wlo0 + k * WIN, lasta)
        return pl.multiple_of(w, 128)

    def fire(k, buf, sem):
        pltpu.async_copy(tabT.at[:, pl.ds(wstart(k), WIN)], buf, sem)

    def drain(buf, sem):
        pltpu.make_async_copy(tabT.at[:, pl.ds(0, WIN)], buf, sem).wait()

    def extract(k, buf):
        wlo = wstart(k)
        whi = wlo + WIN
        gl = jnp.int32(0)
        gh = jnp.int32(0)
        for h in range(ngrp // L):
            mx = gbmax[pl.ds(h * L, L)]
            mn = gbmin[pl.ds(h * L, L)]
            gl = gl + plsc.all_reduce_population_count(mx < wlo)[0]
            gh = gh + plsc.all_reduce_population_count(mn < whi)[0]

        @pl.loop(gl, gh)
        def _(g):
            gids = sids[pl.ds(g * L, L)]
            rel = gids - wlo
            m = jnp.logical_and(gids >= wlo, gids < whi)
            rows = i16 + g * L
            for f in range(NF):
                fvec = jnp.full((L,), f, jnp.int32)
                vals = plsc.load_gather(buf, [fvec, rel], mask=m)
                plsc.store_scatter(cbuf, [rows, fvec], vals, mask=m)

    fire(0, wbuf0, sem0)
    fire(1, wbuf1, sem1)

    @pl.loop(0, nwin2, step=2)
    def _(j):
        drain(wbuf0, sem0)
        extract(j, wbuf0)
        fire(j + 2, wbuf0, sem0)
        drain(wbuf1, sem1)
        extract(j + 1, wbuf1)
        fire(j + 3, wbuf1, sem1)

    drain(wbuf0, sem0)
    drain(wbuf1, sem1)

    pltpu.sync_copy(cbuf, out.at[pl.ds(base, bpw)])


def _scan_body(bpw, su, si, utabT, itabT, uout, iout,
               sids, gbmin, gbmax, cbuf, wbuf0, wbuf1, sem0, sem1):
    wid = lax.axis_index("s") * NC + lax.axis_index("c")
    base = wid * bpw
    pltpu.sync_copy(su.at[pl.ds(base, bpw)], sids)
    _scan_table(utabT, sids, gbmin, gbmax, cbuf, wbuf0, wbuf1,
                sem0, sem1, uout, base, bpw)
    pltpu.sync_copy(si.at[pl.ds(base, bpw)], sids)
    _scan_table(itabT, sids, gbmin, gbmax, cbuf, wbuf0, wbuf1,
                sem0, sem1, iout, base, bpw)


def _sc_scan(su, si, utabT, itabT):
    B = su.shape[0]
    bpw = B // NW
    body = functools.partial(_scan_body, bpw)
    out2 = jax.ShapeDtypeStruct((B, NF), jnp.float32)
    mesh = plsc.VectorSubcoreMesh(
        core_axis_name="c", subcore_axis_name="s", num_cores=NC, num_subcores=NS
    )
    k = pl.kernel(
        body,
        out_type=(out2, out2),
        mesh=mesh,
        compiler_params=pltpu.CompilerParams(needs_layout_passes=False),
        scratch_types=[
            pltpu.VMEM((bpw,), jnp.int32),
            pltpu.VMEM((bpw // L,), jnp.int32),
            pltpu.VMEM((bpw // L,), jnp.int32),
            pltpu.VMEM((bpw, NF), jnp.float32),
            pltpu.VMEM((NF, WIN), jnp.float32),
            pltpu.VMEM((NF, WIN), jnp.float32),
            pltpu.SemaphoreType.DMA,
            pltpu.SemaphoreType.DMA,
        ],
    )
    return k(su, si, utabT, itabT)


def _unperm_body(bpw, semu, semi, invu, invi, uout, iout,
                 uidx, iidx, urows, irows, sem):
    wid = lax.axis_index("s") * NC + lax.axis_index("c")
    nchunk = bpw // 128
    pltpu.sync_copy(invu.at[pl.ds(wid * nchunk, nchunk)], uidx)
    pltpu.sync_copy(invi.at[pl.ds(wid * nchunk, nchunk)], iidx)
    copies = []
    for j in range(nchunk):
        copies.append(pltpu.async_copy(semu.at[uidx.at[j]], urows.at[j], sem))
        copies.append(pltpu.async_copy(semi.at[iidx.at[j]], irows.at[j], sem))
    for c in copies:
        c.wait()
    pltpu.sync_copy(urows, uout.at[pl.ds(wid * nchunk, nchunk)])
    pltpu.sync_copy(irows, iout.at[pl.ds(wid * nchunk, nchunk)])


def _sc_unpermute(semb_u, semb_i, inv_pu, inv_pi):
    B = semb_u.shape[0]
    D = semb_u.shape[1]
    bpw = B // NW
    nchunk = bpw // 128
    invu2 = inv_pu.reshape(B // 128, 128)
    invi2 = inv_pi.reshape(B // 128, 128)
    body = functools.partial(_unperm_body, bpw)
    out3 = jax.ShapeDtypeStruct((B // 128, 128, D), jnp.float32)
    mesh = plsc.VectorSubcoreMesh(
        core_axis_name="c", subcore_axis_name="s", num_cores=NC, num_subcores=NS
    )
    k = pl.kernel(
        body,
        out_type=(out3, out3),
        mesh=mesh,
        compiler_params=pltpu.CompilerParams(use_tc_tiling_on_sc=False),
        scratch_types=[
            pltpu.VMEM((nchunk, 128), jnp.int32),
            pltpu.VMEM((nchunk, 128), jnp.int32),
            pltpu.VMEM((nchunk, 128, D), jnp.float32),
            pltpu.VMEM((nchunk, 128, D), jnp.float32),
            pltpu.SemaphoreType.DMA,
        ],
    )
    u3, i3 = k(semb_u, semb_i, invu2, invi2)
    return u3.reshape(B, D), i3.reshape(B, D)


def _mlp_body(utailo, itailo, u_ref, i_ref, uid_ref, iid_ref, tu_ref, ti_ref,
              w1u_ref, w1i_ref, b1_ref, w2_ref, b2_ref,
              w3_ref, b3_ref, o_ref):
    bt = u_ref.shape[0]

    def fix(x, ids2, tail_ref, tailo):
        n = tail_ref.shape[0]
        idb = lax.broadcast_in_dim(ids2, (bt, n), (0, 1))
        rel = jnp.clip(idb - tailo, 0, n - 1)
        oh = (rel == lax.broadcasted_iota(jnp.int32, (bt, n), 1))
        tv = jnp.dot(oh.astype(jnp.float32), tail_ref[...],
                     preferred_element_type=jnp.float32)
        keep = lax.broadcast_in_dim(ids2 < tailo, (bt, x.shape[1]), (0, 1))
        return jnp.where(keep, x, tv)

    u = fix(u_ref[...], uid_ref[...], tu_ref, utailo)
    i = fix(i_ref[...], iid_ref[...], ti_ref, itailo)
    h = (
        jnp.dot(u, w1u_ref[...], preferred_element_type=jnp.float32)
        + jnp.dot(i, w1i_ref[...], preferred_element_type=jnp.float32)
        + b1_ref[...]
    )
    h = jnp.maximum(h, 0.0)
    h = jnp.dot(h, w2_ref[...], preferred_element_type=jnp.float32) + b2_ref[...]
    h = jnp.maximum(h, 0.0)
    z = jnp.sum(h * w3_ref[...], axis=1) + b3_ref[...]
    o_ref[...] = 1.0 / (1.0 + jnp.exp(-z))


def _tc_mlp(u, i, uid, iid, tail_u, tail_i, W1, b1, W2, b2, W3, b3):
    B, D = u.shape
    BT = 2048
    w1u = W1[:, :D].T
    w1i = W1[:, D:].T
    w2 = W2.T
    w3 = W3[0]
    grid = (B // BT,)
    body = functools.partial(_mlp_body, NUM_USERS_TAILO, NUM_ITEMS_TAILO)
    return pl.pallas_call(
        body,
        grid=grid,
        in_specs=[
            pl.BlockSpec((BT, D), lambda g: (g, 0)),
            pl.BlockSpec((BT, D), lambda g: (g, 0)),
            pl.BlockSpec((BT, 1), lambda g: (g, 0)),
            pl.BlockSpec((BT, 1), lambda g: (g, 0)),
            pl.BlockSpec(tail_u.shape, lambda g: (0, 0)),
            pl.BlockSpec(tail_i.shape, lambda g: (0, 0)),
            pl.BlockSpec(w1u.shape, lambda g: (0, 0)),
            pl.BlockSpec(w1i.shape, lambda g: (0, 0)),
            pl.BlockSpec(b1.shape, lambda g: (0,)),
            pl.BlockSpec(w2.shape, lambda g: (0, 0)),
            pl.BlockSpec(b2.shape, lambda g: (0,)),
            pl.BlockSpec(w3.shape, lambda g: (0,)),
            pl.BlockSpec(b3.shape, lambda g: (0,)),
        ],
        out_specs=pl.BlockSpec((BT,), lambda g: (g,)),
        out_shape=jax.ShapeDtypeStruct((B,), jnp.float32),
    )(u, i, uid.reshape(B, 1), iid.reshape(B, 1), tail_u, tail_i, w1u, w1i, b1, w2, b2, w3, b3)


NUM_USERS_TAILO = 999936
NUM_ITEMS_TAILO = 99968


def kernel(user_ids, item_ids, user_table, item_table, W1, b1, W2, b2, W3, b3):
    B = user_ids.shape[0]
    uid = user_ids.astype(jnp.int32)
    iid = item_ids.astype(jnp.int32)
    pos = lax.iota(jnp.int32, B)
    su, pu = lax.sort((uid, pos), num_keys=1)
    si, pi = lax.sort((iid, pos), num_keys=1)
    _, inv_pu = lax.sort((pu, pos), num_keys=1)
    _, inv_pi = lax.sort((pi, pos), num_keys=1)
    semb_u, semb_i = _sc_scan(su, si, user_table.T, item_table.T)
    u, i = _sc_unpermute(semb_u, semb_i, inv_pu, inv_pi)
    tail_u = user_table[NUM_USERS_TAILO:, :]
    tail_i = item_table[NUM_ITEMS_TAILO:, :]
    return _tc_mlp(u, i, uid, iid, tail_u, tail_i, W1, b1, W2, b2, W3, b3)


# speedup vs baseline: 5.0301x; 1.1130x over previous
import functools

import jax
import jax.numpy as jnp
from jax import lax
from jax.experimental import pallas as pl
from jax.experimental.pallas import tpu as pltpu
from jax.experimental.pallas import tpu_sc as plsc

NC = 2
NS = 16
NW = NC * NS
L = 16
WIN = 768
NF = 32


def _scan_table(tabT, sids, gbmin, gbmax, cbuf, wbuf0, wbuf1,
                sem0, sem1, out, base, bpw):
    n_users = tabT.shape[1]
    lasta = ((n_users - WIN) // 128) * 128
    ngrp = bpw // L

    i16 = lax.iota(jnp.int32, L)
    for h in range(ngrp // L):
        gbmin[pl.ds(h * L, L)] = plsc.load_gather(
            sids, [i16 * L + (h * L * L)])
        gbmax[pl.ds(h * L, L)] = plsc.load_gather(
            sids, [i16 * L + (h * L * L + L - 1)])

    first = sids[pl.ds(0, L)][0]
    last = sids[pl.ds(bpw - L, L)][L - 1]
    wlo0 = jnp.minimum((first >> 7) << 7, lasta)
    nwin = (((last >> 7) << 7) - wlo0) // WIN + 1
    nwin2 = ((nwin + 1) // 2) * 2

    def wstart(k):
        w = jnp.minimum(wlo0 + k * WIN, lasta)
        return pl.multiple_of(w, 128)

    def fire(k, buf, sem):
        pltpu.async_copy(tabT.at[:, pl.ds(wstart(k), WIN)], buf, sem)

    def drain(buf, sem):
        pltpu.make_async_copy(tabT.at[:, pl.ds(0, WIN)], buf, sem).wait()

    def extract(k, buf):
        wlo = wstart(k)
        whi = wlo + WIN
        gl = jnp.int32(0)
        gh = jnp.int32(0)
        for h in range(ngrp // L):
            mx = gbmax[pl.ds(h * L, L)]
            mn = gbmin[pl.ds(h * L, L)]
            gl = gl + plsc.all_reduce_population_count(mx < wlo)[0]
            gh = gh + plsc.all_reduce_population_count(mn < whi)[0]

        @pl.loop(gl, gh)
        def _(g):
            gids = sids[pl.ds(g * L, L)]
            rel = gids - wlo
            m = jnp.logical_and(gids >= wlo, gids < whi)
            rows = i16 + g * L
            for f in range(NF):
                fvec = jnp.full((L,), f, jnp.int32)
                vals = plsc.load_gather(buf, [fvec, rel], mask=m)
                plsc.store_scatter(cbuf, [rows, fvec], vals, mask=m)

    fire(0, wbuf0, sem0)
    fire(1, wbuf1, sem1)

    @pl.loop(0, nwin2, step=2)
    def _(j):
        drain(wbuf0, sem0)
        extract(j, wbuf0)
        fire(j + 2, wbuf0, sem0)
        drain(wbuf1, sem1)
        extract(j + 1, wbuf1)
        fire(j + 3, wbuf1, sem1)

    drain(wbuf0, sem0)
    drain(wbuf1, sem1)

    pltpu.sync_copy(cbuf, out.at[pl.ds(base, bpw)])


def _scan_body(bpw, su, si, utabT, itabT, uout, iout,
               sids, gbmin, gbmax, cbuf, wbuf0, wbuf1, sem0, sem1):
    wid = lax.axis_index("s") * NC + lax.axis_index("c")
    base = wid * bpw
    pltpu.sync_copy(su.at[pl.ds(base, bpw)], sids)
    _scan_table(utabT, sids, gbmin, gbmax, cbuf, wbuf0, wbuf1,
                sem0, sem1, uout, base, bpw)
    pltpu.sync_copy(si.at[pl.ds(base, bpw)], sids)
    _scan_table(itabT, sids, gbmin, gbmax, cbuf, wbuf0, wbuf1,
                sem0, sem1, iout, base, bpw)


def _sc_scan(su, si, utabT, itabT):
    B = su.shape[0]
    bpw = B // NW
    body = functools.partial(_scan_body, bpw)
    out2 = jax.ShapeDtypeStruct((B, 128), jnp.float32)
    mesh = plsc.VectorSubcoreMesh(
        core_axis_name="c", subcore_axis_name="s", num_cores=NC, num_subcores=NS
    )
    k = pl.kernel(
        body,
        out_type=(out2, out2),
        mesh=mesh,
        compiler_params=pltpu.CompilerParams(needs_layout_passes=False),
        scratch_types=[
            pltpu.VMEM((bpw,), jnp.int32),
            pltpu.VMEM((bpw // L,), jnp.int32),
            pltpu.VMEM((bpw // L,), jnp.int32),
            pltpu.VMEM((bpw, 128), jnp.float32),
            pltpu.VMEM((NF, WIN), jnp.float32),
            pltpu.VMEM((NF, WIN), jnp.float32),
            pltpu.SemaphoreType.DMA,
            pltpu.SemaphoreType.DMA,
        ],
    )
    return k(su, si, utabT, itabT)


def _unperm_body(bpw, semu, semi, invu, invi, uout, iout,
                 uidx, iidx, ub0, ub1, ib0, ib1, sem0, sem1):
    wid = lax.axis_index("s") * NC + lax.axis_index("c")
    nchunk = bpw // 128
    pltpu.sync_copy(invu.at[pl.ds(wid * nchunk, nchunk)], uidx)
    pltpu.sync_copy(invi.at[pl.ds(wid * nchunk, nchunk)], iidx)

    def fire(j, ub, ib, sem):
        pltpu.async_copy(semu.at[uidx.at[j]], ub, sem)
        pltpu.async_copy(semi.at[iidx.at[j]], ib, sem)

    def drain(ub, ib, sem):
        pltpu.make_async_copy(semu.at[uidx.at[0]], ub, sem).wait()
        pltpu.make_async_copy(semi.at[iidx.at[0]], ib, sem).wait()

    fire(0, ub0, ib0, sem0)
    fire(1, ub1, ib1, sem1)
    for j in range(nchunk):
        ub, ib, sem = (ub0, ib0, sem0) if j % 2 == 0 else (ub1, ib1, sem1)
        drain(ub, ib, sem)
        row = wid * bpw + j * 128
        pltpu.sync_copy(ub, uout.at[pl.ds(row, 128)])
        pltpu.sync_copy(ib, iout.at[pl.ds(row, 128)])
        if j + 2 < nchunk:
            fire(j + 2, ub, ib, sem)


def _sc_unpermute(semb_u, semb_i, inv_pu, inv_pi):
    B = semb_u.shape[0]
    bpw = B // NW
    nchunk = bpw // 128
    invu2 = inv_pu.reshape(B // 128, 128)
    invi2 = inv_pi.reshape(B // 128, 128)
    body = functools.partial(_unperm_body, bpw)
    out2 = jax.ShapeDtypeStruct((B, 128), jnp.float32)
    mesh = plsc.VectorSubcoreMesh(
        core_axis_name="c", subcore_axis_name="s", num_cores=NC, num_subcores=NS
    )
    k = pl.kernel(
        body,
        out_type=(out2, out2),
        mesh=mesh,
        scratch_types=[
            pltpu.VMEM((nchunk, 128), jnp.int32),
            pltpu.VMEM((nchunk, 128), jnp.int32),
            pltpu.VMEM((128, 128), jnp.float32),
            pltpu.VMEM((128, 128), jnp.float32),
            pltpu.VMEM((128, 128), jnp.float32),
            pltpu.VMEM((128, 128), jnp.float32),
            pltpu.SemaphoreType.DMA,
            pltpu.SemaphoreType.DMA,
        ],
    )
    return k(semb_u, semb_i, invu2, invi2)


def _mlp_body(utailo, itailo, u_ref, i_ref, uid_ref, iid_ref, tu_ref, ti_ref,
              w1u_ref, w1i_ref, b1_ref, w2_ref, b2_ref,
              w3_ref, b3_ref, o_ref):
    bt = u_ref.shape[0]

    def fix(x, ids2, tail_ref, tailo):
        n = tail_ref.shape[0]
        idb = lax.broadcast_in_dim(ids2, (bt, n), (0, 1))
        rel = jnp.clip(idb - tailo, 0, n - 1)
        oh = (rel == lax.broadcasted_iota(jnp.int32, (bt, n), 1))
        tv = jnp.dot(oh.astype(jnp.float32), tail_ref[...],
                     preferred_element_type=jnp.float32)
        keep = lax.broadcast_in_dim(ids2 < tailo, (bt, x.shape[1]), (0, 1))
        return jnp.where(keep, x, tv)

    u = fix(u_ref[:, :32], uid_ref[...], tu_ref, utailo)
    i = fix(i_ref[:, :32], iid_ref[...], ti_ref, itailo)
    h = (
        jnp.dot(u, w1u_ref[...], preferred_element_type=jnp.float32)
        + jnp.dot(i, w1i_ref[...], preferred_element_type=jnp.float32)
        + b1_ref[...]
    )
    h = jnp.maximum(h, 0.0)
    h = jnp.dot(h, w2_ref[...], preferred_element_type=jnp.float32) + b2_ref[...]
    h = jnp.maximum(h, 0.0)
    z = jnp.sum(h * w3_ref[...], axis=1) + b3_ref[...]
    o_ref[...] = 1.0 / (1.0 + jnp.exp(-z))


def _tc_mlp(u, i, uid, iid, tail_u, tail_i, W1, b1, W2, b2, W3, b3):
    B = u.shape[0]
    D = 32
    BT = 2048
    w1u = W1[:, :D].T
    w1i = W1[:, D:].T
    w2 = W2.T
    w3 = W3[0]
    grid = (B // BT,)
    body = functools.partial(_mlp_body, NUM_USERS_TAILO, NUM_ITEMS_TAILO)
    return pl.pallas_call(
        body,
        grid=grid,
        in_specs=[
            pl.BlockSpec((BT, 128), lambda g: (g, 0)),
            pl.BlockSpec((BT, 128), lambda g: (g, 0)),
            pl.BlockSpec((BT, 1), lambda g: (g, 0)),
            pl.BlockSpec((BT, 1), lambda g: (g, 0)),
            pl.BlockSpec(tail_u.shape, lambda g: (0, 0)),
            pl.BlockSpec(tail_i.shape, lambda g: (0, 0)),
            pl.BlockSpec(w1u.shape, lambda g: (0, 0)),
            pl.BlockSpec(w1i.shape, lambda g: (0, 0)),
            pl.BlockSpec(b1.shape, lambda g: (0,)),
            pl.BlockSpec(w2.shape, lambda g: (0, 0)),
            pl.BlockSpec(b2.shape, lambda g: (0,)),
            pl.BlockSpec(w3.shape, lambda g: (0,)),
            pl.BlockSpec(b3.shape, lambda g: (0,)),
        ],
        out_specs=pl.BlockSpec((BT,), lambda g: (g,)),
        out_shape=jax.ShapeDtypeStruct((B,), jnp.float32),
    )(u, i, uid.reshape(B, 1), iid.reshape(B, 1), tail_u, tail_i, w1u, w1i, b1, w2, b2, w3, b3)


NUM_USERS_TAILO = 999936
NUM_ITEMS_TAILO = 99968


def kernel(user_ids, item_ids, user_table, item_table, W1, b1, W2, b2, W3, b3):
    B = user_ids.shape[0]
    uid = user_ids.astype(jnp.int32)
    iid = item_ids.astype(jnp.int32)
    pos = lax.iota(jnp.int32, B)
    su, pu = lax.sort((uid, pos), num_keys=1)
    si, pi = lax.sort((iid, pos), num_keys=1)
    _, inv_pu = lax.sort((pu, pos), num_keys=1)
    _, inv_pi = lax.sort((pi, pos), num_keys=1)
    semb_u, semb_i = _sc_scan(su, si, user_table.T, item_table.T)
    u, i = _sc_unpermute(semb_u, semb_i, inv_pu, inv_pi)
    tail_u = user_table[NUM_USERS_TAILO:, :]
    tail_i = item_table[NUM_ITEMS_TAILO:, :]
    return _tc_mlp(u, i, uid, iid, tail_u, tail_i, W1, b1, W2, b2, W3, b3)


# speedup vs baseline: 5.0440x; 1.0028x over previous
import functools

import jax
import jax.numpy as jnp
from jax import lax
from jax.experimental import pallas as pl
from jax.experimental.pallas import tpu as pltpu
from jax.experimental.pallas import tpu_sc as plsc

NC = 2
NS = 16
NW = NC * NS
L = 16
WIN = 768
NF = 32


def _scan_table(tabT, sids, gbmin, gbmax, cbuf, wbuf0, wbuf1,
                sem0, sem1, out, base, bpw):
    n_users = tabT.shape[1]
    lasta = ((n_users - WIN) // 128) * 128
    ngrp = bpw // L

    i16 = lax.iota(jnp.int32, L)
    for h in range(ngrp // L):
        gbmin[pl.ds(h * L, L)] = plsc.load_gather(
            sids, [i16 * L + (h * L * L)])
        gbmax[pl.ds(h * L, L)] = plsc.load_gather(
            sids, [i16 * L + (h * L * L + L - 1)])

    first = sids[pl.ds(0, L)][0]
    last = sids[pl.ds(bpw - L, L)][L - 1]
    wlo0 = jnp.minimum((first >> 7) << 7, lasta)
    nwin = (((last >> 7) << 7) - wlo0) // WIN + 1
    nwin2 = ((nwin + 1) // 2) * 2

    def wstart(k):
        w = jnp.minimum(wlo0 + k * WIN, lasta)
        return pl.multiple_of(w, 128)

    def fire(k, buf, sem):
        pltpu.async_copy(tabT.at[:, pl.ds(wstart(k), WIN)], buf, sem)

    def drain(buf, sem):
        pltpu.make_async_copy(tabT.at[:, pl.ds(0, WIN)], buf, sem).wait()

    def extract(k, buf):
        wlo = wstart(k)
        whi = wlo + WIN
        gl = jnp.int32(0)
        gh = jnp.int32(0)
        for h in range(ngrp // L):
            mx = gbmax[pl.ds(h * L, L)]
            mn = gbmin[pl.ds(h * L, L)]
            gl = gl + plsc.all_reduce_population_count(mx < wlo)[0]
            gh = gh + plsc.all_reduce_population_count(mn < whi)[0]

        @pl.loop(gl, gh)
        def _(g):
            gids = sids[pl.ds(g * L, L)]
            rel = gids - wlo
            m = jnp.logical_and(gids >= wlo, gids < whi)
            rows = i16 + g * L
            for f in range(NF):
                fvec = jnp.full((L,), f, jnp.int32)
                vals = plsc.load_gather(buf, [fvec, rel], mask=m)
                plsc.store_scatter(cbuf, [rows, fvec], vals, mask=m)

    fire(0, wbuf0, sem0)
    fire(1, wbuf1, sem1)

    @pl.loop(0, nwin2, step=2)
    def _(j):
        drain(wbuf0, sem0)
        extract(j, wbuf0)
        fire(j + 2, wbuf0, sem0)
        drain(wbuf1, sem1)
        extract(j + 1, wbuf1)
        fire(j + 3, wbuf1, sem1)

    drain(wbuf0, sem0)
    drain(wbuf1, sem1)

    pltpu.sync_copy(cbuf, out.at[pl.ds(base, bpw)])


def _scan_body(bpw, su, si, utabT, itabT, uout, iout,
               sids, gbmin, gbmax, cbuf, wbuf0, wbuf1, sem0, sem1):
    wid = lax.axis_index("s") * NC + lax.axis_index("c")
    base = wid * bpw
    pltpu.sync_copy(su.at[pl.ds(base, bpw)], sids)
    _scan_table(utabT, sids, gbmin, gbmax, cbuf, wbuf0, wbuf1,
                sem0, sem1, uout, base, bpw)
    pltpu.sync_copy(si.at[pl.ds(base, bpw)], sids)
    _scan_table(itabT, sids, gbmin, gbmax, cbuf, wbuf0, wbuf1,
                sem0, sem1, iout, base, bpw)


def _sc_scan(su, si, utabT, itabT):
    B = su.shape[0]
    bpw = B // NW
    body = functools.partial(_scan_body, bpw)
    out2 = jax.ShapeDtypeStruct((B, 128), jnp.float32)
    mesh = plsc.VectorSubcoreMesh(
        core_axis_name="c", subcore_axis_name="s", num_cores=NC, num_subcores=NS
    )
    k = pl.kernel(
        body,
        out_type=(out2, out2),
        mesh=mesh,
        compiler_params=pltpu.CompilerParams(needs_layout_passes=False),
        scratch_types=[
            pltpu.VMEM((bpw,), jnp.int32),
            pltpu.VMEM((bpw // L,), jnp.int32),
            pltpu.VMEM((bpw // L,), jnp.int32),
            pltpu.VMEM((bpw, 128), jnp.float32),
            pltpu.VMEM((NF, WIN), jnp.float32),
            pltpu.VMEM((NF, WIN), jnp.float32),
            pltpu.SemaphoreType.DMA,
            pltpu.SemaphoreType.DMA,
        ],
    )
    return k(su, si, utabT, itabT)


def _unperm_body(bpw, semu, semi, invu, invi, uout, iout,
                 uidx, iidx, ub0, ub1, ib0, ib1, sem0, sem1):
    wid = lax.axis_index("s") * NC + lax.axis_index("c")
    nchunk = bpw // 128
    pltpu.sync_copy(invu.at[pl.ds(wid * nchunk, nchunk)], uidx)
    pltpu.sync_copy(invi.at[pl.ds(wid * nchunk, nchunk)], iidx)

    def fire(j, ub, ib, sem):
        pltpu.async_copy(semu.at[uidx.at[j]], ub, sem)
        pltpu.async_copy(semi.at[iidx.at[j]], ib, sem)

    def drain(ub, ib, sem):
        pltpu.make_async_copy(semu.at[uidx.at[0]], ub, sem).wait()
        pltpu.make_async_copy(semi.at[iidx.at[0]], ib, sem).wait()

    fire(0, ub0, ib0, sem0)
    fire(1, ub1, ib1, sem1)
    for j in range(nchunk):
        ub, ib, sem = (ub0, ib0, sem0) if j % 2 == 0 else (ub1, ib1, sem1)
        drain(ub, ib, sem)
        row = wid * bpw + j * 128
        pltpu.sync_copy(ub, uout.at[pl.ds(row, 128)])
        pltpu.sync_copy(ib, iout.at[pl.ds(row, 128)])
        if j + 2 < nchunk:
            fire(j + 2, ub, ib, sem)


def _sc_unpermute(semb_u, semb_i, inv_pu, inv_pi):
    B = semb_u.shape[0]
    bpw = B // NW
    nchunk = bpw // 128
    invu2 = inv_pu.reshape(B // 128, 128)
    invi2 = inv_pi.reshape(B // 128, 128)
    body = functools.partial(_unperm_body, bpw)
    out2 = jax.ShapeDtypeStruct((B, 128), jnp.float32)
    mesh = plsc.VectorSubcoreMesh(
        core_axis_name="c", subcore_axis_name="s", num_cores=NC, num_subcores=NS
    )
    k = pl.kernel(
        body,
        out_type=(out2, out2),
        mesh=mesh,
        scratch_types=[
            pltpu.VMEM((nchunk, 128), jnp.int32),
            pltpu.VMEM((nchunk, 128), jnp.int32),
            pltpu.VMEM((128, 128), jnp.float32),
            pltpu.VMEM((128, 128), jnp.float32),
            pltpu.VMEM((128, 128), jnp.float32),
            pltpu.VMEM((128, 128), jnp.float32),
            pltpu.SemaphoreType.DMA,
            pltpu.SemaphoreType.DMA,
        ],
    )
    return k(semb_u, semb_i, invu2, invi2)


def _mlp_body(utailo, itailo, u_ref, i_ref, uid_ref, iid_ref, tu_ref, ti_ref,
              w1u_ref, w1i_ref, b1_ref, w2_ref, b2_ref,
              w3_ref, b3_ref, o_ref):
    bt = u_ref.shape[0]

    def fix(x, ids2, tail_ref, tailo):
        n = tail_ref.shape[0]
        idb = lax.broadcast_in_dim(ids2, (bt, n), (0, 1))
        rel = jnp.clip(idb - tailo, 0, n - 1)
        oh = (rel == lax.broadcasted_iota(jnp.int32, (bt, n), 1))
        tv = jnp.dot(oh.astype(jnp.float32), tail_ref[...],
                     preferred_element_type=jnp.float32)
        keep = lax.broadcast_in_dim(ids2 < tailo, (bt, x.shape[1]), (0, 1))
        return jnp.where(keep, x, tv)

    u = fix(u_ref[:, :32], uid_ref[...], tu_ref, utailo)
    i = fix(i_ref[:, :32], iid_ref[...], ti_ref, itailo)
    h = (
        jnp.dot(u, w1u_ref[...], preferred_element_type=jnp.float32)
        + jnp.dot(i, w1i_ref[...], preferred_element_type=jnp.float32)
        + b1_ref[...]
    )
    h = jnp.maximum(h, 0.0)
    h = jnp.dot(h, w2_ref[...], preferred_element_type=jnp.float32) + b2_ref[...]
    h = jnp.maximum(h, 0.0)
    z = jnp.sum(h * w3_ref[...], axis=1) + b3_ref[...]
    o_ref[...] = 1.0 / (1.0 + jnp.exp(-z))


def _tc_mlp(u, i, uid, iid, tail_u, tail_i, W1, b1, W2, b2, W3, b3):
    B = u.shape[0]
    D = 32
    BT = 4096
    w1u = W1[:, :D].T
    w1i = W1[:, D:].T
    w2 = W2.T
    w3 = W3[0]
    grid = (B // BT,)
    body = functools.partial(_mlp_body, NUM_USERS_TAILO, NUM_ITEMS_TAILO)
    return pl.pallas_call(
        body,
        grid=grid,
        in_specs=[
            pl.BlockSpec((BT, 128), lambda g: (g, 0)),
            pl.BlockSpec((BT, 128), lambda g: (g, 0)),
            pl.BlockSpec((BT, 1), lambda g: (g, 0)),
            pl.BlockSpec((BT, 1), lambda g: (g, 0)),
            pl.BlockSpec(tail_u.shape, lambda g: (0, 0)),
            pl.BlockSpec(tail_i.shape, lambda g: (0, 0)),
            pl.BlockSpec(w1u.shape, lambda g: (0, 0)),
            pl.BlockSpec(w1i.shape, lambda g: (0, 0)),
            pl.BlockSpec(b1.shape, lambda g: (0,)),
            pl.BlockSpec(w2.shape, lambda g: (0, 0)),
            pl.BlockSpec(b2.shape, lambda g: (0,)),
            pl.BlockSpec(w3.shape, lambda g: (0,)),
            pl.BlockSpec(b3.shape, lambda g: (0,)),
        ],
        out_specs=pl.BlockSpec((BT,), lambda g: (g,)),
        out_shape=jax.ShapeDtypeStruct((B,), jnp.float32),
    )(u, i, uid.reshape(B, 1), iid.reshape(B, 1), tail_u, tail_i, w1u, w1i, b1, w2, b2, w3, b3)


NUM_USERS_TAILO = 999936
NUM_ITEMS_TAILO = 99968


def kernel(user_ids, item_ids, user_table, item_table, W1, b1, W2, b2, W3, b3):
    B = user_ids.shape[0]
    uid = user_ids.astype(jnp.int32)
    iid = item_ids.astype(jnp.int32)
    pos = lax.iota(jnp.int32, B)
    su, pu = lax.sort((uid, pos), num_keys=1)
    si, pi = lax.sort((iid, pos), num_keys=1)
    _, inv_pu = lax.sort((pu, pos), num_keys=1)
    _, inv_pi = lax.sort((pi, pos), num_keys=1)
    semb_u, semb_i = _sc_scan(su, si, user_table.T, item_table.T)
    u, i = _sc_unpermute(semb_u, semb_i, inv_pu, inv_pi)
    tail_u = user_table[NUM_USERS_TAILO:, :]
    tail_i = item_table[NUM_ITEMS_TAILO:, :]
    return _tc_mlp(u, i, uid, iid, tail_u, tail_i, W1, b1, W2, b2, W3, b3)


# speedup vs baseline: 5.1610x; 1.0232x over previous
import functools

import jax
import jax.numpy as jnp
from jax import lax
from jax.experimental import pallas as pl
from jax.experimental.pallas import tpu as pltpu
from jax.experimental.pallas import tpu_sc as plsc

NC = 2
NS = 16
NW = NC * NS
L = 16
WIN = 512
NF = 32


def _scan_table(tabT, sids, gbmin, gbmax, cbuf, wbuf0, wbuf1, wbuf2,
                sem0, sem1, sem2, out, base, bpw):
    n_users = tabT.shape[1]
    lasta = ((n_users - WIN) // 128) * 128
    ngrp = bpw // L

    i16 = lax.iota(jnp.int32, L)
    for h in range(ngrp // L):
        gbmin[pl.ds(h * L, L)] = plsc.load_gather(
            sids, [i16 * L + (h * L * L)])
        gbmax[pl.ds(h * L, L)] = plsc.load_gather(
            sids, [i16 * L + (h * L * L + L - 1)])

    first = sids[pl.ds(0, L)][0]
    last = sids[pl.ds(bpw - L, L)][L - 1]
    wlo0 = jnp.minimum((first >> 7) << 7, lasta)
    nwin = (((last >> 7) << 7) - wlo0) // WIN + 1
    nwin3 = ((nwin + 2) // 3) * 3

    def wstart(k):
        w = jnp.minimum(wlo0 + k * WIN, lasta)
        return pl.multiple_of(w, 128)

    def fire(k, buf, sem):
        pltpu.async_copy(tabT.at[:, pl.ds(wstart(k), WIN)], buf, sem)

    def drain(buf, sem):
        pltpu.make_async_copy(tabT.at[:, pl.ds(0, WIN)], buf, sem).wait()

    def extract(k, buf):
        wlo = wstart(k)
        whi = wlo + WIN
        gl = jnp.int32(0)
        gh = jnp.int32(0)
        for h in range(ngrp // L):
            mx = gbmax[pl.ds(h * L, L)]
            mn = gbmin[pl.ds(h * L, L)]
            gl = gl + plsc.all_reduce_population_count(mx < wlo)[0]
            gh = gh + plsc.all_reduce_population_count(mn < whi)[0]

        @pl.loop(gl, gh)
        def _(g):
            gids = sids[pl.ds(g * L, L)]
            rel = gids - wlo
            m = jnp.logical_and(gids >= wlo, gids < whi)
            rows = i16 + g * L
            for f in range(NF):
                fvec = jnp.full((L,), f, jnp.int32)
                vals = plsc.load_gather(buf, [fvec, rel], mask=m)
                plsc.store_scatter(cbuf, [rows, fvec], vals, mask=m)

    fire(0, wbuf0, sem0)
    fire(1, wbuf1, sem1)
    fire(2, wbuf2, sem2)

    @pl.loop(0, nwin3, step=3)
    def _(j):
        drain(wbuf0, sem0)
        extract(j, wbuf0)
        fire(j + 3, wbuf0, sem0)
        drain(wbuf1, sem1)
        extract(j + 1, wbuf1)
        fire(j + 4, wbuf1, sem1)
        drain(wbuf2, sem2)
        extract(j + 2, wbuf2)
        fire(j + 5, wbuf2, sem2)

    drain(wbuf0, sem0)
    drain(wbuf1, sem1)
    drain(wbuf2, sem2)

    pltpu.sync_copy(cbuf, out.at[pl.ds(base, bpw)])


def _scan_body(bpw, su, si, utabT, itabT, uout, iout,
               sids, gbmin, gbmax, cbuf, wbuf0, wbuf1, wbuf2,
               sem0, sem1, sem2):
    wid = lax.axis_index("s") * NC + lax.axis_index("c")
    base = wid * bpw
    pltpu.sync_copy(su.at[pl.ds(base, bpw)], sids)
    _scan_table(utabT, sids, gbmin, gbmax, cbuf, wbuf0, wbuf1, wbuf2,
                sem0, sem1, sem2, uout, base, bpw)
    pltpu.sync_copy(si.at[pl.ds(base, bpw)], sids)
    _scan_table(itabT, sids, gbmin, gbmax, cbuf, wbuf0, wbuf1, wbuf2,
                sem0, sem1, sem2, iout, base, bpw)


def _sc_scan(su, si, utabT, itabT):
    B = su.shape[0]
    bpw = B // NW
    body = functools.partial(_scan_body, bpw)
    out2 = jax.ShapeDtypeStruct((B, 128), jnp.float32)
    mesh = plsc.VectorSubcoreMesh(
        core_axis_name="c", subcore_axis_name="s", num_cores=NC, num_subcores=NS
    )
    k = pl.kernel(
        body,
        out_type=(out2, out2),
        mesh=mesh,
        compiler_params=pltpu.CompilerParams(needs_layout_passes=False),
        scratch_types=[
            pltpu.VMEM((bpw,), jnp.int32),
            pltpu.VMEM((bpw // L,), jnp.int32),
            pltpu.VMEM((bpw // L,), jnp.int32),
            pltpu.VMEM((bpw, 128), jnp.float32),
            pltpu.VMEM((NF, WIN), jnp.float32),
            pltpu.VMEM((NF, WIN), jnp.float32),
            pltpu.VMEM((NF, WIN), jnp.float32),
            pltpu.SemaphoreType.DMA,
            pltpu.SemaphoreType.DMA,
            pltpu.SemaphoreType.DMA,
        ],
    )
    return k(su, si, utabT, itabT)


def _unperm_body(bpw, semu, semi, invu, invi, uout, iout,
                 uidx, iidx, ub0, ub1, ib0, ib1, sem0, sem1, ws0, ws1):
    wid = lax.axis_index("s") * NC + lax.axis_index("c")
    nchunk = bpw // 128
    pltpu.sync_copy(invu.at[pl.ds(wid * nchunk, nchunk)], uidx)
    pltpu.sync_copy(invi.at[pl.ds(wid * nchunk, nchunk)], iidx)

    def slot(j):
        return (ub0, ib0, sem0, ws0) if j % 2 == 0 else (ub1, ib1, sem1, ws1)

    def fire_g(j):
        ub, ib, sem, _ = slot(j)
        pltpu.async_copy(semu.at[uidx.at[j]], ub, sem)
        pltpu.async_copy(semi.at[iidx.at[j]], ib, sem)

    def drain_g(j):
        ub, ib, sem, _ = slot(j)
        pltpu.make_async_copy(semu.at[uidx.at[0]], ub, sem).wait()
        pltpu.make_async_copy(semi.at[iidx.at[0]], ib, sem).wait()

    def fire_w(j):
        ub, ib, _, ws = slot(j)
        row = wid * bpw + j * 128
        pltpu.async_copy(ub, uout.at[pl.ds(row, 128)], ws)
        pltpu.async_copy(ib, iout.at[pl.ds(row, 128)], ws)

    def drain_w(j):
        ub, ib, _, ws = slot(j)
        row = wid * bpw + j * 128
        pltpu.make_async_copy(ub, uout.at[pl.ds(row, 128)], ws).wait()
        pltpu.make_async_copy(ib, iout.at[pl.ds(row, 128)], ws).wait()

    fire_g(0)
    fire_g(1)
    for j in range(nchunk):
        drain_g(j)
        fire_w(j)
        if j + 2 < nchunk:
            drain_w(j)
            fire_g(j + 2)
    drain_w(nchunk - 2)
    drain_w(nchunk - 1)


def _sc_unpermute(semb_u, semb_i, inv_pu, inv_pi):
    B = semb_u.shape[0]
    bpw = B // NW
    nchunk = bpw // 128
    invu2 = inv_pu.reshape(B // 128, 128)
    invi2 = inv_pi.reshape(B // 128, 128)
    body = functools.partial(_unperm_body, bpw)
    out2 = jax.ShapeDtypeStruct((B, 128), jnp.float32)
    mesh = plsc.VectorSubcoreMesh(
        core_axis_name="c", subcore_axis_name="s", num_cores=NC, num_subcores=NS
    )
    k = pl.kernel(
        body,
        out_type=(out2, out2),
        mesh=mesh,
        scratch_types=[
            pltpu.VMEM((nchunk, 128), jnp.int32),
            pltpu.VMEM((nchunk, 128), jnp.int32),
            pltpu.VMEM((128, 128), jnp.float32),
            pltpu.VMEM((128, 128), jnp.float32),
            pltpu.VMEM((128, 128), jnp.float32),
            pltpu.VMEM((128, 128), jnp.float32),
            pltpu.SemaphoreType.DMA,
            pltpu.SemaphoreType.DMA,
            pltpu.SemaphoreType.DMA,
            pltpu.SemaphoreType.DMA,
        ],
    )
    return k(semb_u, semb_i, invu2, invi2)


def _mlp_body(utailo, itailo, u_ref, i_ref, uid_ref, iid_ref, tu_ref, ti_ref,
              w1u_ref, w1i_ref, b1_ref, w2_ref, b2_ref,
              w3_ref, b3_ref, o_ref):
    bt = u_ref.shape[0]

    def fix(x, ids2, tail_ref, tailo):
        n = tail_ref.shape[0]
        idb = lax.broadcast_in_dim(ids2, (bt, n), (0, 1))
        rel = jnp.clip(idb - tailo, 0, n - 1)
        oh = (rel == lax.broadcasted_iota(jnp.int32, (bt, n), 1))
        tv = jnp.dot(oh.astype(jnp.float32), tail_ref[...],
                     preferred_element_type=jnp.float32)
        keep = lax.broadcast_in_dim(ids2 < tailo, (bt, x.shape[1]), (0, 1))
        return jnp.where(keep, x, tv)

    u = fix(u_ref[:, :32], uid_ref[...], tu_ref, utailo)
    i = fix(i_ref[:, :32], iid_ref[...], ti_ref, itailo)
    h = (
        jnp.dot(u, w1u_ref[...], preferred_element_type=jnp.float32)
        + jnp.dot(i, w1i_ref[...], preferred_element_type=jnp.float32)
        + b1_ref[...]
    )
    h = jnp.maximum(h, 0.0)
    h = jnp.dot(h, w2_ref[...], preferred_element_type=jnp.float32) + b2_ref[...]
    h = jnp.maximum(h, 0.0)
    z = jnp.sum(h * w3_ref[...], axis=1) + b3_ref[...]
    o_ref[...] = 1.0 / (1.0 + jnp.exp(-z))


def _tc_mlp(u, i, uid, iid, tail_u, tail_i, W1, b1, W2, b2, W3, b3):
    B = u.shape[0]
    D = 32
    BT = 4096
    w1u = W1[:, :D].T
    w1i = W1[:, D:].T
    w2 = W2.T
    w3 = W3[0]
    grid = (B // BT,)
    body = functools.partial(_mlp_body, NUM_USERS_TAILO, NUM_ITEMS_TAILO)
    return pl.pallas_call(
        body,
        grid=grid,
        in_specs=[
            pl.BlockSpec((BT, 128), lambda g: (g, 0)),
            pl.BlockSpec((BT, 128), lambda g: (g, 0)),
            pl.BlockSpec((BT, 1), lambda g: (g, 0)),
            pl.BlockSpec((BT, 1), lambda g: (g, 0)),
            pl.BlockSpec(tail_u.shape, lambda g: (0, 0)),
            pl.BlockSpec(tail_i.shape, lambda g: (0, 0)),
            pl.BlockSpec(w1u.shape, lambda g: (0, 0)),
            pl.BlockSpec(w1i.shape, lambda g: (0, 0)),
            pl.BlockSpec(b1.shape, lambda g: (0,)),
            pl.BlockSpec(w2.shape, lambda g: (0, 0)),
            pl.BlockSpec(b2.shape, lambda g: (0,)),
            pl.BlockSpec(w3.shape, lambda g: (0,)),
            pl.BlockSpec(b3.shape, lambda g: (0,)),
        ],
        out_specs=pl.BlockSpec((BT,), lambda g: (g,)),
        out_shape=jax.ShapeDtypeStruct((B,), jnp.float32),
    )(u, i, uid.reshape(B, 1), iid.reshape(B, 1), tail_u, tail_i, w1u, w1i, b1, w2, b2, w3, b3)


NUM_USERS_TAILO = 999936
NUM_ITEMS_TAILO = 99968


def kernel(user_ids, item_ids, user_table, item_table, W1, b1, W2, b2, W3, b3):
    B = user_ids.shape[0]
    uid = user_ids.astype(jnp.int32)
    iid = item_ids.astype(jnp.int32)
    pos = lax.iota(jnp.int32, B)
    su, pu = lax.sort((uid, pos), num_keys=1)
    si, pi = lax.sort((iid, pos), num_keys=1)
    _, inv_pu = lax.sort((pu, pos), num_keys=1)
    _, inv_pi = lax.sort((pi, pos), num_keys=1)
    semb_u, semb_i = _sc_scan(su, si, user_table.T, item_table.T)
    u, i = _sc_unpermute(semb_u, semb_i, inv_pu, inv_pi)
    tail_u = user_table[NUM_USERS_TAILO:, :]
    tail_i = item_table[NUM_ITEMS_TAILO:, :]
    return _tc_mlp(u, i, uid, iid, tail_u, tail_i, W1, b1, W2, b2, W3, b3)


# speedup vs baseline: 5.1660x; 1.0010x over previous
import functools

import jax
import jax.numpy as jnp
from jax import lax
from jax.experimental import pallas as pl
from jax.experimental.pallas import tpu as pltpu
from jax.experimental.pallas import tpu_sc as plsc

NC = 2
NS = 16
NW = NC * NS
L = 16
WIN = 512
NF = 32


def _scan_table(tabT, sids, gbmin, gbmax, cbuf, wbuf0, wbuf1, wbuf2,
                sem0, sem1, sem2, out, base, bpw):
    n_users = tabT.shape[1]
    lasta = ((n_users - WIN) // 128) * 128
    ngrp = bpw // L

    i16 = lax.iota(jnp.int32, L)
    for h in range(ngrp // L):
        gbmin[pl.ds(h * L, L)] = plsc.load_gather(
            sids, [i16 * L + (h * L * L)])
        gbmax[pl.ds(h * L, L)] = plsc.load_gather(
            sids, [i16 * L + (h * L * L + L - 1)])

    first = sids[pl.ds(0, L)][0]
    last = sids[pl.ds(bpw - L, L)][L - 1]
    wlo0 = jnp.minimum((first >> 7) << 7, lasta)
    nwin = (((last >> 7) << 7) - wlo0) // WIN + 1
    nwin3 = ((nwin + 2) // 3) * 3

    def wstart(k):
        w = jnp.minimum(wlo0 + k * WIN, lasta)
        return pl.multiple_of(w, 128)

    def fire(k, buf, sem):
        pltpu.async_copy(tabT.at[:, pl.ds(wstart(k), WIN)], buf, sem)

    def drain(buf, sem):
        pltpu.make_async_copy(tabT.at[:, pl.ds(0, WIN)], buf, sem).wait()

    def extract(k, buf):
        wlo = wstart(k)
        whi = wlo + WIN
        gl = jnp.int32(0)
        gh = jnp.int32(0)
        for h in range(ngrp // L):
            mx = gbmax[pl.ds(h * L, L)]
            mn = gbmin[pl.ds(h * L, L)]
            gl = gl + plsc.all_reduce_population_count(mx < wlo)[0]
            gh = gh + plsc.all_reduce_population_count(mn < whi)[0]

        @pl.loop(gl, gh)
        def _(g):
            gids = sids[pl.ds(g * L, L)]
            rel = gids - wlo
            m = jnp.logical_and(gids >= wlo, gids < whi)
            rows = i16 + g * L
            for f in range(NF):
                fvec = jnp.full((L,), f, jnp.int32)
                vals = plsc.load_gather(buf, [fvec, rel], mask=m)
                plsc.store_scatter(cbuf, [rows, fvec], vals, mask=m)

    fire(0, wbuf0, sem0)
    fire(1, wbuf1, sem1)
    fire(2, wbuf2, sem2)

    @pl.loop(0, nwin3, step=3)
    def _(j):
        drain(wbuf0, sem0)
        extract(j, wbuf0)
        fire(j + 3, wbuf0, sem0)
        drain(wbuf1, sem1)
        extract(j + 1, wbuf1)
        fire(j + 4, wbuf1, sem1)
        drain(wbuf2, sem2)
        extract(j + 2, wbuf2)
        fire(j + 5, wbuf2, sem2)

    drain(wbuf0, sem0)
    drain(wbuf1, sem1)
    drain(wbuf2, sem2)

    pltpu.sync_copy(cbuf, out.at[pl.ds(base, bpw)])


def _scan_body(bpw, su, si, utabT, itabT, uout, iout,
               sids, gbmin, gbmax, cbuf, wbuf0, wbuf1, wbuf2,
               sem0, sem1, sem2):
    wid = lax.axis_index("s") * NC + lax.axis_index("c")
    base = wid * bpw
    pltpu.sync_copy(su.at[pl.ds(base, bpw)], sids)
    _scan_table(utabT, sids, gbmin, gbmax, cbuf, wbuf0, wbuf1, wbuf2,
                sem0, sem1, sem2, uout, base, bpw)
    pltpu.sync_copy(si.at[pl.ds(base, bpw)], sids)
    _scan_table(itabT, sids, gbmin, gbmax, cbuf, wbuf0, wbuf1, wbuf2,
                sem0, sem1, sem2, iout, base, bpw)


def _sc_scan(su, si, utabT, itabT):
    B = su.shape[0]
    bpw = B // NW
    body = functools.partial(_scan_body, bpw)
    out2 = jax.ShapeDtypeStruct((B, 128), jnp.float32)
    mesh = plsc.VectorSubcoreMesh(
        core_axis_name="c", subcore_axis_name="s", num_cores=NC, num_subcores=NS
    )
    k = pl.kernel(
        body,
        out_type=(out2, out2),
        mesh=mesh,
        compiler_params=pltpu.CompilerParams(needs_layout_passes=False),
        scratch_types=[
            pltpu.VMEM((bpw,), jnp.int32),
            pltpu.VMEM((bpw // L,), jnp.int32),
            pltpu.VMEM((bpw // L,), jnp.int32),
            pltpu.VMEM((bpw, 128), jnp.float32),
            pltpu.VMEM((NF, WIN), jnp.float32),
            pltpu.VMEM((NF, WIN), jnp.float32),
            pltpu.VMEM((NF, WIN), jnp.float32),
            pltpu.SemaphoreType.DMA,
            pltpu.SemaphoreType.DMA,
            pltpu.SemaphoreType.DMA,
        ],
    )
    return k(su, si, utabT, itabT)


def _unperm_body(bpw, semu, semi, invu, invi, uout, iout,
                 uidx, iidx, ub0, ub1, ib0, ib1, sem0, sem1):
    wid = lax.axis_index("s") * NC + lax.axis_index("c")
    nchunk = bpw // 128
    pltpu.sync_copy(invu.at[pl.ds(wid * nchunk, nchunk)], uidx)
    pltpu.sync_copy(invi.at[pl.ds(wid * nchunk, nchunk)], iidx)

    def fire(j, ub, ib, sem):
        pltpu.async_copy(semu.at[uidx.at[j]], ub, sem)
        pltpu.async_copy(semi.at[iidx.at[j]], ib, sem)

    def drain(ub, ib, sem):
        pltpu.make_async_copy(semu.at[uidx.at[0]], ub, sem).wait()
        pltpu.make_async_copy(semi.at[iidx.at[0]], ib, sem).wait()

    fire(0, ub0, ib0, sem0)
    fire(1, ub1, ib1, sem1)
    for j in range(nchunk):
        ub, ib, sem = (ub0, ib0, sem0) if j % 2 == 0 else (ub1, ib1, sem1)
        drain(ub, ib, sem)
        row = wid * bpw + j * 128
        pltpu.sync_copy(ub, uout.at[pl.ds(row, 128)])
        pltpu.sync_copy(ib, iout.at[pl.ds(row, 128)])
        if j + 2 < nchunk:
            fire(j + 2, ub, ib, sem)


def _sc_unpermute(semb_u, semb_i, inv_pu, inv_pi):
    B = semb_u.shape[0]
    bpw = B // NW
    nchunk = bpw // 128
    invu2 = inv_pu.reshape(B // 128, 128)
    invi2 = inv_pi.reshape(B // 128, 128)
    body = functools.partial(_unperm_body, bpw)
    out2 = jax.ShapeDtypeStruct((B, 128), jnp.float32)
    mesh = plsc.VectorSubcoreMesh(
        core_axis_name="c", subcore_axis_name="s", num_cores=NC, num_subcores=NS
    )
    k = pl.kernel(
        body,
        out_type=(out2, out2),
        mesh=mesh,
        scratch_types=[
            pltpu.VMEM((nchunk, 128), jnp.int32),
            pltpu.VMEM((nchunk, 128), jnp.int32),
            pltpu.VMEM((128, 128), jnp.float32),
            pltpu.VMEM((128, 128), jnp.float32),
            pltpu.VMEM((128, 128), jnp.float32),
            pltpu.VMEM((128, 128), jnp.float32),
            pltpu.SemaphoreType.DMA,
            pltpu.SemaphoreType.DMA,
        ],
    )
    return k(semb_u, semb_i, invu2, invi2)


def _mlp_body(utailo, itailo, u_ref, i_ref, uid_ref, iid_ref, tu_ref, ti_ref,
              w1u_ref, w1i_ref, b1_ref, w2_ref, b2_ref,
              w3_ref, b3_ref, o_ref):
    bt = u_ref.shape[0]

    def fix(x, ids2, tail_ref, tailo):
        n = tail_ref.shape[0]
        idb = lax.broadcast_in_dim(ids2, (bt, n), (0, 1))
        rel = jnp.clip(idb - tailo, 0, n - 1)
        oh = (rel == lax.broadcasted_iota(jnp.int32, (bt, n), 1))
        tv = jnp.dot(oh.astype(jnp.float32), tail_ref[...],
                     preferred_element_type=jnp.float32)
        keep = lax.broadcast_in_dim(ids2 < tailo, (bt, x.shape[1]), (0, 1))
        return jnp.where(keep, x, tv)

    u = fix(u_ref[:, :32], uid_ref[...], tu_ref, utailo)
    i = fix(i_ref[:, :32], iid_ref[...], ti_ref, itailo)
    h = (
        jnp.dot(u, w1u_ref[...], preferred_element_type=jnp.float32)
        + jnp.dot(i, w1i_ref[...], preferred_element_type=jnp.float32)
        + b1_ref[...]
    )
    h = jnp.maximum(h, 0.0)
    h = jnp.dot(h, w2_ref[...], preferred_element_type=jnp.float32) + b2_ref[...]
    h = jnp.maximum(h, 0.0)
    z = jnp.sum(h * w3_ref[...], axis=1) + b3_ref[...]
    o_ref[...] = 1.0 / (1.0 + jnp.exp(-z))


def _tc_mlp(u, i, uid, iid, tail_u, tail_i, W1, b1, W2, b2, W3, b3):
    B = u.shape[0]
    D = 32
    BT = 4096
    w1u = W1[:, :D].T
    w1i = W1[:, D:].T
    w2 = W2.T
    w3 = W3[0]
    grid = (B // BT,)
    body = functools.partial(_mlp_body, NUM_USERS_TAILO, NUM_ITEMS_TAILO)
    return pl.pallas_call(
        body,
        grid=grid,
        in_specs=[
            pl.BlockSpec((BT, 128), lambda g: (g, 0)),
            pl.BlockSpec((BT, 128), lambda g: (g, 0)),
            pl.BlockSpec((BT, 1), lambda g: (g, 0)),
            pl.BlockSpec((BT, 1), lambda g: (g, 0)),
            pl.BlockSpec(tail_u.shape, lambda g: (0, 0)),
            pl.BlockSpec(tail_i.shape, lambda g: (0, 0)),
            pl.BlockSpec(w1u.shape, lambda g: (0, 0)),
            pl.BlockSpec(w1i.shape, lambda g: (0, 0)),
            pl.BlockSpec(b1.shape, lambda g: (0,)),
            pl.BlockSpec(w2.shape, lambda g: (0, 0)),
            pl.BlockSpec(b2.shape, lambda g: (0,)),
            pl.BlockSpec(w3.shape, lambda g: (0,)),
            pl.BlockSpec(b3.shape, lambda g: (0,)),
        ],
        out_specs=pl.BlockSpec((BT,), lambda g: (g,)),
        out_shape=jax.ShapeDtypeStruct((B,), jnp.float32),
    )(u, i, uid.reshape(B, 1), iid.reshape(B, 1), tail_u, tail_i, w1u, w1i, b1, w2, b2, w3, b3)


NUM_USERS_TAILO = 999936
NUM_ITEMS_TAILO = 99968


def kernel(user_ids, item_ids, user_table, item_table, W1, b1, W2, b2, W3, b3):
    B = user_ids.shape[0]
    uid = user_ids.astype(jnp.int32)
    iid = item_ids.astype(jnp.int32)
    pos = lax.iota(jnp.int32, B)
    su, pu = lax.sort((uid, pos), num_keys=1)
    si, pi = lax.sort((iid, pos), num_keys=1)
    _, inv_pu = lax.sort((pu, pos), num_keys=1)
    _, inv_pi = lax.sort((pi, pos), num_keys=1)
    semb_u, semb_i = _sc_scan(su, si, user_table.T, item_table.T)
    u, i = _sc_unpermute(semb_u, semb_i, inv_pu, inv_pi)
    tail_u = user_table[NUM_USERS_TAILO:, :]
    tail_i = item_table[NUM_ITEMS_TAILO:, :]
    return _tc_mlp(u, i, uid, iid, tail_u, tail_i, W1, b1, W2, b2, W3, b3)
